# baseline (speedup 1.0000x reference)
import functools

import jax
import jax.numpy as jnp
from jax import lax
from jax.experimental import pallas as pl
from jax.experimental.pallas import tpu as pltpu
from jax.experimental.pallas import tpu_sc as plsc

VOCAB = 100000
EMB = 30
B = 4096
L = 50
NC = 2
NS = 16
NW = NC * NS
SPW = B // NW
CC = 16
NCHUNK = SPW // CC
NROW = CC * L
NVIEW = NROW
VROWS = VOCAB * 4
GSUB = 80
NGSUB = NVIEW // GSUB
OUTW = 128
HID = 128


def _sc_pool_body(i_ct, i_cd, i_tt, i_td, T0, T1, T2, T3, out,
                  idx_v, idxg, rows_v, out_v, sems):
  wid = lax.axis_index("s") * NC + lax.axis_index("c")
  lanes = lax.iota(jnp.int32, CC)
  lane_out = lanes * OUTW
  zeros16 = jnp.zeros((CC,), jnp.float32)

  for t in range(4):
    for c in (EMB, EMB + 1):
      plsc.store_scatter(out_v, [lane_out + (t * 32 + c)], zeros16)

  idx_arrays = (i_ct, i_cd, i_tt, i_td)
  tables = (T0, T1, T2, T3)

  def build_fire(g, t, p):
    base_w = (wid * SPW + g * CC) * L
    pltpu.sync_copy(idx_arrays[t].at[pl.ds(base_w, NROW)], idx_v.at[p])

    def build_body(mi, c2):
      v = idx_v[p, pl.ds(mi * CC, CC)]
      a = v * 4 + jnp.bitwise_and(v, 3)
      plsc.store_scatter(idxg.at[p], [mi * CC + lanes], a)
      return c2

    lax.fori_loop(0, NROW // CC, build_body, 0)
    for sgi in range(NGSUB):
      pltpu.async_copy(
          tables[t].at[idxg.at[p].at[pl.ds(sgi * GSUB, GSUB)]],
          rows_v.at[p].at[pl.ds(sgi * GSUB, GSUB)],
          sems.at[p])

  def drain(p):
    pltpu.make_async_copy(
        tables[0].at[pl.ds(0, NVIEW)], rows_v.at[p], sems.at[p]).wait()

  def sum_unit(t, p):
    def j_body(j, accs):
      row = lanes * L + j
      out_accs = []
      for k in range(EMB):
        col = jnp.full((CC,), k, jnp.int32)
        val = plsc.load_gather(rows_v.at[p], [row, col])
        out_accs.append(accs[k] + val)
      return tuple(out_accs)

    accs = lax.fori_loop(0, L, j_body, (zeros16,) * EMB)
    for k in range(EMB):
      plsc.store_scatter(out_v, [lane_out + (t * 32 + k)], accs[k])

  build_fire(0, 0, 0)

  def chunk_body(g, carry):
    base_s = wid * SPW + g * CC
    for t in range(4):
      if t < 3:
        build_fire(g, t + 1, (t + 1) & 1)
      else:
        @pl.when(g < NCHUNK - 1)
        def _():
          build_fire(g + 1, 0, 0)
      drain(t & 1)
      sum_unit(t, t & 1)
    pltpu.sync_copy(out_v, out.at[pl.ds(base_s * OUTW, CC * OUTW)])
    return carry

  lax.fori_loop(0, NCHUNK, chunk_body, 0)


@functools.cache
def _sc_pool():
  mesh = plsc.VectorSubcoreMesh(core_axis_name="c", subcore_axis_name="s")
  return pl.kernel(
      _sc_pool_body,
      out_type=jax.ShapeDtypeStruct((B * OUTW,), jnp.float32),
      mesh=mesh,
      scratch_types=[
          pltpu.VMEM((2, NROW), jnp.int32),
          pltpu.VMEM((2, NVIEW), jnp.int32),
          pltpu.VMEM((2, NVIEW, 32), jnp.float32),
          pltpu.VMEM((CC * OUTW,), jnp.float32),
          pltpu.SemaphoreType.DMA((2,)),
      ],
      compiler_params=pltpu.CompilerParams(
          needs_layout_passes=False, use_tc_tiling_on_sc=False),
  )


PB = 5000


def _pad_body(x_ref, o_ref):
  x = x_ref[...]
  z2 = jnp.zeros((PB, 2), jnp.float32)
  x32 = jnp.concatenate([x, z2], axis=1)
  q = jax.lax.broadcasted_iota(jnp.int32, (PB, 1), 0) % 4
  quarters = [jnp.where(q == i, x32, 0.0) for i in range(4)]
  o_ref[...] = jnp.concatenate(quarters, axis=1)


_padder = pl.pallas_call(
    _pad_body,
    grid=(VOCAB // PB,),
    in_specs=[pl.BlockSpec((PB, EMB), lambda i: (i, 0))],
    out_specs=pl.BlockSpec((PB, 128), lambda i: (i, 0)),
    out_shape=jax.ShapeDtypeStruct((VOCAB, 128), jnp.float32),
)


def _mlp_body(x_ref, w1_ref, b1_ref, w2_ref, b2_ref, w3_ref, b3_ref, o_ref):
  h = jnp.maximum(x_ref[...], 0.0)
  h = jnp.dot(h, w1_ref[...], preferred_element_type=jnp.float32) + b1_ref[...]
  h = jnp.maximum(h, 0.0)
  h = jnp.dot(h, w2_ref[...], preferred_element_type=jnp.float32) + b2_ref[...]
  h = jnp.maximum(h, 0.0)
  o = jnp.dot(h, w3_ref[...], preferred_element_type=jnp.float32) + b3_ref[...]
  o_ref[...] = 1.0 / (1.0 + jnp.exp(-o))


BM = 512

_mlp = pl.pallas_call(
    _mlp_body,
    grid=(B // BM,),
    in_specs=[
        pl.BlockSpec((BM, OUTW), lambda i: (i, 0)),
        pl.BlockSpec((OUTW, HID), lambda i: (0, 0)),
        pl.BlockSpec((1, HID), lambda i: (0, 0)),
        pl.BlockSpec((HID, HID), lambda i: (0, 0)),
        pl.BlockSpec((1, HID), lambda i: (0, 0)),
        pl.BlockSpec((HID, 8), lambda i: (0, 0)),
        pl.BlockSpec((1, 8), lambda i: (0, 0)),
    ],
    out_specs=pl.BlockSpec((BM, 8), lambda i: (i, 0)),
    out_shape=jax.ShapeDtypeStruct((B, 8), jnp.float32),
)


def kernel(content_title, content_description, topic_title, topic_description,
           T_ct, T_cd, T_tt, T_td, W1, b1, W2, b2, W3, b3):
  def prep_idx(a):
    return a.astype(jnp.int32).reshape(B * L)

  def prep_tab(t):
    return _padder(t).reshape(VROWS, 32)

  pooled = _sc_pool()(
      prep_idx(content_title), prep_idx(content_description),
      prep_idx(topic_title), prep_idx(topic_description),
      prep_tab(T_ct), prep_tab(T_cd), prep_tab(T_tt), prep_tab(T_td)
  ).reshape(B, OUTW)

  w1 = W1.reshape(4, EMB, 30)
  w1 = jnp.pad(w1, ((0, 0), (0, 2), (0, 0))).reshape(4 * 32, 30)
  W1p = jnp.pad(w1, ((0, 0), (0, HID - 30)))
  b1p = jnp.pad(b1, (0, HID - 30)).reshape(1, HID)
  W2p = jnp.pad(W2, ((0, HID - 30), (0, HID - 30)))
  b2p = jnp.pad(b2, (0, HID - 30)).reshape(1, HID)
  W3p = jnp.pad(W3, ((0, HID - 30), (0, 7)))
  b3p = jnp.pad(b3, (0, 7)).reshape(1, 8)

  out8 = _mlp(pooled, W1p, b1p, W2p, b2p, W3p, b3p)
  return out8[:, :1]


# speedup vs baseline: 1.9383x; 1.9383x over previous
import functools

import jax
import jax.numpy as jnp
from jax import lax
from jax.experimental import pallas as pl
from jax.experimental.pallas import tpu as pltpu
from jax.experimental.pallas import tpu_sc as plsc

VOCAB = 100000
EMB = 30
B = 4096
L = 50
NC = 2
NS = 16
NW = NC * NS
SPW = B // NW
CC = 16
NCHUNK = SPW // CC
NROW = CC * L
NVIEW = 3 * NROW
VROWS = VOCAB * EMB // 16
GSUB = 120
NGSUB = NVIEW // GSUB
OUTW = 128
HID = 128


def _sc_pool_body(i_ct, i_cd, i_tt, i_td, T0, T1, T2, T3, out,
                  idx_v, idxg, dbase, rows_v, out_v, sems):
  wid = lax.axis_index("s") * NC + lax.axis_index("c")
  lanes = lax.iota(jnp.int32, CC)
  lane_out = lanes * OUTW
  zeros16 = jnp.zeros((CC,), jnp.float32)

  for t in range(4):
    for c in (EMB, EMB + 1):
      plsc.store_scatter(out_v, [lane_out + (t * 32 + c)], zeros16)

  idx_arrays = (i_ct, i_cd, i_tt, i_td)
  tables = (T0, T1, T2, T3)

  def build_fire(g, t, p):
    base_w = (wid * SPW + g * CC) * L
    pltpu.sync_copy(idx_arrays[t].at[pl.ds(base_w, NROW)], idx_v.at[p])

    def build_body(mi, c2):
      v = idx_v[p, pl.ds(mi * CC, CC)]
      flat0 = v * 30
      a = lax.shift_right_logical(flat0, 4)
      r = jnp.bitwise_and(flat0, 15)
      a2 = jnp.minimum(a + 2, VROWS - 1)
      pos = mi * (3 * CC) + lanes * 3
      plsc.store_scatter(idxg.at[p], [pos], a)
      plsc.store_scatter(idxg.at[p], [pos + 1], a + 1)
      plsc.store_scatter(idxg.at[p], [pos + 2], a2)
      dvec = (mi * CC + lanes) * 48 + r
      plsc.store_scatter(dbase.at[p], [mi * CC + lanes], dvec)
      return c2

    lax.fori_loop(0, NROW // CC, build_body, 0)
    for sgi in range(NGSUB):
      pltpu.async_copy(
          tables[t].at[idxg.at[p].at[pl.ds(sgi * GSUB, GSUB)]],
          rows_v.at[p].at[pl.ds(sgi * GSUB, GSUB)],
          sems.at[p])

  def drain(p):
    pltpu.make_async_copy(
        tables[0].at[pl.ds(0, NVIEW)], rows_v.at[p], sems.at[p]).wait()

  def sum_unit(t, p):
    def j_body(j, accs):
      dvec = plsc.load_gather(dbase.at[p], [lanes * L + j])
      out_accs = []
      for k in range(EMB):
        w = dvec + k
        val = plsc.load_gather(
            rows_v.at[p],
            [lax.shift_right_logical(w, 4), jnp.bitwise_and(w, 15)])
        out_accs.append(accs[k] + val)
      return tuple(out_accs)

    accs = lax.fori_loop(0, L, j_body, (zeros16,) * EMB)
    for k in range(EMB):
      plsc.store_scatter(out_v, [lane_out + (t * 32 + k)], accs[k])

  build_fire(0, 0, 0)

  def chunk_body(g, carry):
    base_s = wid * SPW + g * CC
    for t in range(4):
      if t < 3:
        build_fire(g, t + 1, (t + 1) & 1)
      else:
        @pl.when(g < NCHUNK - 1)
        def _():
          build_fire(g + 1, 0, 0)
      drain(t & 1)
      sum_unit(t, t & 1)
    pltpu.sync_copy(out_v, out.at[pl.ds(base_s * OUTW, CC * OUTW)])
    return carry

  lax.fori_loop(0, NCHUNK, chunk_body, 0)


@functools.cache
def _sc_pool():
  mesh = plsc.VectorSubcoreMesh(core_axis_name="c", subcore_axis_name="s")
  return pl.kernel(
      _sc_pool_body,
      out_type=jax.ShapeDtypeStruct((B * OUTW,), jnp.float32),
      mesh=mesh,
      scratch_types=[
          pltpu.VMEM((2, NROW), jnp.int32),
          pltpu.VMEM((2, NVIEW), jnp.int32),
          pltpu.VMEM((2, NROW), jnp.int32),
          pltpu.VMEM((2, NVIEW, 16), jnp.float32),
          pltpu.VMEM((CC * OUTW,), jnp.float32),
          pltpu.SemaphoreType.DMA((2,)),
      ],
      compiler_params=pltpu.CompilerParams(
          needs_layout_passes=False, use_tc_tiling_on_sc=False),
  )


def _mlp_body(x_ref, w1_ref, b1_ref, w2_ref, b2_ref, w3_ref, b3_ref, o_ref):
  h = jnp.maximum(x_ref[...], 0.0)
  h = jnp.dot(h, w1_ref[...], preferred_element_type=jnp.float32) + b1_ref[...]
  h = jnp.maximum(h, 0.0)
  h = jnp.dot(h, w2_ref[...], preferred_element_type=jnp.float32) + b2_ref[...]
  h = jnp.maximum(h, 0.0)
  o = jnp.dot(h, w3_ref[...], preferred_element_type=jnp.float32) + b3_ref[...]
  o_ref[...] = 1.0 / (1.0 + jnp.exp(-o))


BM = 512

_mlp = pl.pallas_call(
    _mlp_body,
    grid=(B // BM,),
    in_specs=[
        pl.BlockSpec((BM, OUTW), lambda i: (i, 0)),
        pl.BlockSpec((OUTW, HID), lambda i: (0, 0)),
        pl.BlockSpec((1, HID), lambda i: (0, 0)),
        pl.BlockSpec((HID, HID), lambda i: (0, 0)),
        pl.BlockSpec((1, HID), lambda i: (0, 0)),
        pl.BlockSpec((HID, 8), lambda i: (0, 0)),
        pl.BlockSpec((1, 8), lambda i: (0, 0)),
    ],
    out_specs=pl.BlockSpec((BM, 8), lambda i: (i, 0)),
    out_shape=jax.ShapeDtypeStruct((B, 8), jnp.float32),
)


def kernel(content_title, content_description, topic_title, topic_description,
           T_ct, T_cd, T_tt, T_td, W1, b1, W2, b2, W3, b3):
  def prep_idx(a):
    return a.astype(jnp.int32).reshape(B * L)

  def prep_tab(t):
    return t.reshape(VROWS, 16)

  pooled = _sc_pool()(
      prep_idx(content_title), prep_idx(content_description),
      prep_idx(topic_title), prep_idx(topic_description),
      prep_tab(T_ct), prep_tab(T_cd), prep_tab(T_tt), prep_tab(T_td)
  ).reshape(B, OUTW)

  w1 = W1.reshape(4, EMB, 30)
  w1 = jnp.pad(w1, ((0, 0), (0, 2), (0, 0))).reshape(4 * 32, 30)
  W1p = jnp.pad(w1, ((0, 0), (0, HID - 30)))
  b1p = jnp.pad(b1, (0, HID - 30)).reshape(1, HID)
  W2p = jnp.pad(W2, ((0, HID - 30), (0, HID - 30)))
  b2p = jnp.pad(b2, (0, HID - 30)).reshape(1, HID)
  W3p = jnp.pad(W3, ((0, HID - 30), (0, 7)))
  b3p = jnp.pad(b3, (0, 7)).reshape(1, 8)

  out8 = _mlp(pooled, W1p, b1p, W2p, b2p, W3p, b3p)
  return out8[:, :1]
